# Initial kernel scaffold; baseline (speedup 1.0000x reference)
import jax
import jax.numpy as jnp
from jax.experimental import pallas as pl


def kernel(segment_ids, table):
    raise NotImplementedError("write your pallas kernel here")


# speedup vs baseline: 1.0419x; 1.0419x over previous
import jax
import jax.numpy as jnp
from jax.experimental import pallas as pl
from jax.experimental.pallas import tpu as pltpu
from jax.experimental.pallas import tpu_sc as plsc

_WINDOW = 128


def kernel(segment_ids, table):
    batch, hist = segment_ids.shape
    num_rows, dim = table.shape
    total = batch * hist

    flat_idx = segment_ids.reshape(1, total).astype(jnp.int32)

    mesh = plsc.VectorSubcoreMesh(core_axis_name="c", subcore_axis_name="s")

    @pl.kernel(
        out_type=jax.ShapeDtypeStruct((total, dim), table.dtype),
        mesh=mesh,
        compiler_params=pltpu.CompilerParams(use_tc_tiling_on_sc=False),
    )
    def gather_kernel(table_hbm, idx_hbm, out_hbm):
        def body(i_vmem, o_vmem):
            pltpu.sync_copy(table_hbm.at[i_vmem.at[0]], o_vmem)

        pltpu.emit_pipeline(
            body,
            grid=(total // _WINDOW,),
            in_specs=[
                pl.BlockSpec((1, _WINDOW), index_map=lambda i: (0, i))
            ],
            out_specs=[
                pl.BlockSpec((_WINDOW, dim), index_map=lambda i: (i, 0))
            ],
            core_axis_name=("c", "s"),
            dimension_semantics=(pltpu.PARALLEL,),
        )(idx_hbm, out_hbm)

    out = gather_kernel(table, flat_idx)
    return out.reshape(batch, hist, dim)


# speedup vs baseline: 1.7915x; 1.7194x over previous
import jax
import jax.numpy as jnp
from jax import lax
from jax.experimental import pallas as pl
from jax.experimental.pallas import tpu as pltpu
from jax.experimental.pallas import tpu_sc as plsc

_NC = 2
_NS = 16
_NW = _NC * _NS
_NBUF = 8


def kernel(segment_ids, table):
    batch, hist = segment_ids.shape
    num_rows, dim = table.shape
    rows_per_w = batch // _NW
    assert batch % _NW == 0 and rows_per_w % _NBUF == 0

    idx = segment_ids.astype(jnp.int32)
    mesh = plsc.VectorSubcoreMesh(core_axis_name="c", subcore_axis_name="s")

    @pl.kernel(
        out_type=jax.ShapeDtypeStruct((batch, hist, dim), table.dtype),
        mesh=mesh,
        scratch_types=[
            pltpu.VMEM((rows_per_w, hist), jnp.int32),
            pltpu.VMEM((_NBUF, hist, dim), jnp.float32),
            pltpu.SemaphoreType.DMA((_NBUF,)),
            pltpu.SemaphoreType.DMA((_NBUF,)),
        ],
        compiler_params=pltpu.CompilerParams(use_tc_tiling_on_sc=False),
    )
    def gather_kernel(table_hbm, idx_hbm, out_hbm, idx_v, rows_v, gsem, wsem):
        wid = lax.axis_index("s") * _NC + lax.axis_index("c")
        base = wid * rows_per_w
        pltpu.sync_copy(idx_hbm.at[pl.ds(base, rows_per_w)], idx_v)

        def fire_gather(r, b):
            pltpu.async_copy(
                table_hbm.at[idx_v.at[r]],
                rows_v.at[b],
                gsem.at[b],
            )

        for b in range(_NBUF):
            fire_gather(b, b)

        @pl.loop(0, rows_per_w, step=_NBUF)
        def _(r0):
            for b in range(_NBUF):
                r = r0 + b
                pltpu.make_async_copy(
                    table_hbm.at[idx_v.at[r]],
                    rows_v.at[b],
                    gsem.at[b],
                ).wait()
                wb = pltpu.async_copy(
                    rows_v.at[b],
                    out_hbm.at[base + r],
                    wsem.at[b],
                )

                @pl.when(r + _NBUF < rows_per_w)
                def _():
                    wb.wait()
                    fire_gather(r + _NBUF, b)

        for b in range(_NBUF):
            pltpu.make_async_copy(
                rows_v.at[b],
                out_hbm.at[base],
                wsem.at[b],
            ).wait()

    return gather_kernel(table, idx)
